# initial kernel scaffold (unmeasured)
import jax
import jax.numpy as jnp
from jax import lax
from jax.experimental import pallas as pl
from jax.experimental.pallas import tpu as pltpu

N_DEV = 16
D = 512
ROWS = D // N_DEV


def kernel(partial, resid, gamma):
    def body(x_ref, resid_ref, gamma_ref, out_ref,
             stage_ref, recv_ref,
             send_sems, recv_sems, send_sems2, recv_sems2):
        my = lax.axis_index("i")

        stage_ref[...] = x_ref[0].astype(jnp.bfloat16).reshape(N_DEV, ROWS, D)

        s1 = []
        for j in range(N_DEV):
            rdma = pltpu.make_async_remote_copy(
                src_ref=stage_ref.at[j],
                dst_ref=recv_ref.at[my],
                send_sem=send_sems.at[j],
                recv_sem=recv_sems.at[my],
                device_id=(j,),
                device_id_type=pl.DeviceIdType.MESH,
            )
            s1.append(rdma)

        for j in range(N_DEV):
            @pl.when(my != j)
            def _(rdma=s1[j]):
                rdma.start()

        recv_ref[pl.ds(my, 1)] = stage_ref[pl.ds(my, 1)]

        for s in range(N_DEV):
            @pl.when(my != s)
            def _(rdma=s1[s]):
                w = pltpu.make_async_remote_copy(
                    src_ref=stage_ref.at[s],
                    dst_ref=recv_ref.at[s],
                    send_sem=send_sems.at[s],
                    recv_sem=recv_sems.at[s],
                    device_id=(s,),
                    device_id_type=pl.DeviceIdType.MESH,
                )
                w.wait_recv()

        acc = jnp.sum(recv_ref[...].astype(jnp.float32), axis=0)
        y = acc + resid_ref[pl.ds(my * ROWS, ROWS), :]
        ms = jnp.mean(y * y, axis=-1, keepdims=True)
        out_chunk = y * lax.rsqrt(ms + 1e-6) * gamma_ref[:][None, :]
        out_ref[pl.ds(my * ROWS, ROWS), :] = out_chunk

        s2 = []
        for j in range(N_DEV):
            rdma = pltpu.make_async_remote_copy(
                src_ref=out_ref.at[pl.ds(my * ROWS, ROWS)],
                dst_ref=out_ref.at[pl.ds(my * ROWS, ROWS)],
                send_sem=send_sems2.at[j],
                recv_sem=recv_sems2.at[my],
                device_id=(j,),
                device_id_type=pl.DeviceIdType.MESH,
            )
            s2.append(rdma)

        for j in range(N_DEV):
            @pl.when(my != j)
            def _(rdma=s2[j]):
                rdma.start()

        for j in range(N_DEV):
            @pl.when(my != j)
            def _(rdma=s1[j]):
                rdma.wait_send()

        for s in range(N_DEV):
            @pl.when(my != s)
            def _(s_=s):
                w = pltpu.make_async_remote_copy(
                    src_ref=out_ref.at[pl.ds(s_ * ROWS, ROWS)],
                    dst_ref=out_ref.at[pl.ds(s_ * ROWS, ROWS)],
                    send_sem=send_sems2.at[s_],
                    recv_sem=recv_sems2.at[s_],
                    device_id=(s_,),
                    device_id_type=pl.DeviceIdType.MESH,
                )
                w.wait_recv()
        for j in range(N_DEV):
            @pl.when(my != j)
            def _(rdma=s2[j]):
                rdma.wait_send()

    return pl.pallas_call(
        body,
        out_shape=jax.ShapeDtypeStruct((D, D), jnp.float32),
        in_specs=[
            pl.BlockSpec(memory_space=pltpu.VMEM),
            pl.BlockSpec(memory_space=pltpu.VMEM),
            pl.BlockSpec(memory_space=pltpu.VMEM),
        ],
        out_specs=pl.BlockSpec(memory_space=pltpu.VMEM),
        scratch_shapes=[
            pltpu.VMEM((N_DEV, ROWS, D), jnp.bfloat16),
            pltpu.VMEM((N_DEV, ROWS, D), jnp.bfloat16),
            pltpu.SemaphoreType.DMA((N_DEV,)),
            pltpu.SemaphoreType.DMA((N_DEV,)),
            pltpu.SemaphoreType.DMA((N_DEV,)),
            pltpu.SemaphoreType.DMA((N_DEV,)),
        ],
        compiler_params=pltpu.CompilerParams(collective_id=0),
    )(partial, resid, gamma)


# baseline (device time: 31058 ns/iter reference)
import jax
import jax.numpy as jnp
from jax import lax
from jax.experimental import pallas as pl
from jax.experimental.pallas import tpu as pltpu

N_DEV = 16
D = 512
ROWS = D // N_DEV


def kernel(partial, resid, gamma):
    def body(x_ref, resid_ref, gamma_ref, out_ref,
             stage_ref, recv_ref,
             send_sems, recv_sems, send_sems2, recv_sems2):
        my = lax.axis_index("i")

        stage_ref[...] = x_ref[0].astype(jnp.bfloat16).reshape(N_DEV, ROWS, D)

        s1 = []
        for j in range(N_DEV):
            rdma = pltpu.make_async_remote_copy(
                src_ref=stage_ref.at[j],
                dst_ref=recv_ref.at[my],
                send_sem=send_sems.at[j],
                recv_sem=recv_sems.at[my],
                device_id=(j,),
                device_id_type=pl.DeviceIdType.MESH,
            )
            s1.append(rdma)

        for j in range(N_DEV):
            @pl.when(my != j)
            def _(rdma=s1[j]):
                rdma.start()

        recv_ref[pl.ds(my, 1)] = stage_ref[pl.ds(my, 1)]

        for s in range(N_DEV):
            @pl.when(my != s)
            def _(rdma=s1[s]):
                w = pltpu.make_async_remote_copy(
                    src_ref=stage_ref.at[s],
                    dst_ref=recv_ref.at[s],
                    send_sem=send_sems.at[s],
                    recv_sem=recv_sems.at[s],
                    device_id=(s,),
                    device_id_type=pl.DeviceIdType.MESH,
                )
                w.wait_recv()

        acc = jnp.sum(recv_ref[...].astype(jnp.float32), axis=0)
        y = acc + resid_ref[pl.ds(my * ROWS, ROWS), :]
        ms = jnp.mean(y * y, axis=-1, keepdims=True)
        out_chunk = y * lax.rsqrt(ms + 1e-6) * gamma_ref[:][None, :]
        out_ref[pl.ds(my * ROWS, ROWS), :] = out_chunk

        s2 = []
        for j in range(N_DEV):
            rdma = pltpu.make_async_remote_copy(
                src_ref=out_ref.at[pl.ds(my * ROWS, ROWS)],
                dst_ref=out_ref.at[pl.ds(my * ROWS, ROWS)],
                send_sem=send_sems2.at[j],
                recv_sem=recv_sems2.at[my],
                device_id=(j,),
                device_id_type=pl.DeviceIdType.MESH,
            )
            s2.append(rdma)

        for j in range(N_DEV):
            @pl.when(my != j)
            def _(rdma=s2[j]):
                rdma.start()

        for j in range(N_DEV):
            @pl.when(my != j)
            def _(rdma=s1[j]):
                rdma.wait_send()

        for s in range(N_DEV):
            @pl.when(my != s)
            def _(s_=s):
                w = pltpu.make_async_remote_copy(
                    src_ref=out_ref.at[pl.ds(s_ * ROWS, ROWS)],
                    dst_ref=out_ref.at[pl.ds(s_ * ROWS, ROWS)],
                    send_sem=send_sems2.at[s_],
                    recv_sem=recv_sems2.at[s_],
                    device_id=(s_,),
                    device_id_type=pl.DeviceIdType.MESH,
                )
                w.wait_recv()
        for j in range(N_DEV):
            @pl.when(my != j)
            def _(rdma=s2[j]):
                rdma.wait_send()

    return pl.pallas_call(
        body,
        out_shape=jax.ShapeDtypeStruct((D, D), jnp.float32),
        in_specs=[
            pl.BlockSpec(memory_space=pltpu.VMEM),
            pl.BlockSpec(memory_space=pltpu.VMEM),
            pl.BlockSpec(memory_space=pltpu.VMEM),
        ],
        out_specs=pl.BlockSpec(memory_space=pltpu.VMEM),
        scratch_shapes=[
            pltpu.VMEM((N_DEV, ROWS, D), jnp.bfloat16),
            pltpu.VMEM((N_DEV, ROWS, D), jnp.bfloat16),
            pltpu.SemaphoreType.DMA((N_DEV,)),
            pltpu.SemaphoreType.DMA((N_DEV,)),
            pltpu.SemaphoreType.DMA((N_DEV,)),
            pltpu.SemaphoreType.DMA((N_DEV,)),
        ],
    )(partial, resid, gamma)


# device time: 27591 ns/iter; 1.1257x vs baseline; 1.1257x over previous
import jax
import jax.numpy as jnp
from jax import lax
from jax.experimental import pallas as pl
from jax.experimental.pallas import tpu as pltpu

N_DEV = 16
D = 512
ROWS = D // N_DEV


def kernel(partial, resid, gamma):
    def body(x_ref, resid_ref, gamma_ref, out_ref,
             stage_ref, recv_ref, gather_ref,
             send_sems, recv_sems, send_sems2, recv_sems2):
        my = lax.axis_index("i")

        stage_ref[...] = x_ref[0].astype(jnp.bfloat16).reshape(N_DEV, ROWS, D)

        s1 = []
        for j in range(N_DEV):
            rdma = pltpu.make_async_remote_copy(
                src_ref=stage_ref.at[j],
                dst_ref=recv_ref.at[my],
                send_sem=send_sems.at[j],
                recv_sem=recv_sems.at[my],
                device_id=(j,),
                device_id_type=pl.DeviceIdType.MESH,
            )
            s1.append(rdma)

        for j in range(N_DEV):
            @pl.when(my != j)
            def _(rdma=s1[j]):
                rdma.start()

        recv_ref[pl.ds(my, 1)] = stage_ref[pl.ds(my, 1)]

        for s in range(N_DEV):
            @pl.when(my != s)
            def _(rdma=s1[s]):
                w = pltpu.make_async_remote_copy(
                    src_ref=stage_ref.at[s],
                    dst_ref=recv_ref.at[s],
                    send_sem=send_sems.at[s],
                    recv_sem=recv_sems.at[s],
                    device_id=(s,),
                    device_id_type=pl.DeviceIdType.MESH,
                )
                w.wait_recv()

        acc = jnp.sum(recv_ref[...].astype(jnp.float32), axis=0)
        y = acc + resid_ref[pl.ds(my * ROWS, ROWS), :]
        ms = jnp.mean(y * y, axis=-1, keepdims=True)
        out_chunk = y * lax.rsqrt(ms + 1e-6) * gamma_ref[:][None, :]
        gather_ref[pl.ds(my, 1)] = out_chunk.astype(jnp.bfloat16)[None]

        s2 = []
        for j in range(N_DEV):
            rdma = pltpu.make_async_remote_copy(
                src_ref=gather_ref.at[my],
                dst_ref=gather_ref.at[my],
                send_sem=send_sems2.at[j],
                recv_sem=recv_sems2.at[my],
                device_id=(j,),
                device_id_type=pl.DeviceIdType.MESH,
            )
            s2.append(rdma)

        for j in range(N_DEV):
            @pl.when(my != j)
            def _(rdma=s2[j]):
                rdma.start()

        for j in range(N_DEV):
            @pl.when(my != j)
            def _(rdma=s1[j]):
                rdma.wait_send()

        for s in range(N_DEV):
            @pl.when(my != s)
            def _(s_=s):
                w = pltpu.make_async_remote_copy(
                    src_ref=gather_ref.at[s_],
                    dst_ref=gather_ref.at[s_],
                    send_sem=send_sems2.at[s_],
                    recv_sem=recv_sems2.at[s_],
                    device_id=(s_,),
                    device_id_type=pl.DeviceIdType.MESH,
                )
                w.wait_recv()
        for j in range(N_DEV):
            @pl.when(my != j)
            def _(rdma=s2[j]):
                rdma.wait_send()

        out_ref[...] = gather_ref[...].astype(jnp.float32).reshape(D, D)

    return pl.pallas_call(
        body,
        out_shape=jax.ShapeDtypeStruct((D, D), jnp.float32),
        in_specs=[
            pl.BlockSpec(memory_space=pltpu.VMEM),
            pl.BlockSpec(memory_space=pltpu.VMEM),
            pl.BlockSpec(memory_space=pltpu.VMEM),
        ],
        out_specs=pl.BlockSpec(memory_space=pltpu.VMEM),
        scratch_shapes=[
            pltpu.VMEM((N_DEV, ROWS, D), jnp.bfloat16),
            pltpu.VMEM((N_DEV, ROWS, D), jnp.bfloat16),
            pltpu.VMEM((N_DEV, ROWS, D), jnp.bfloat16),
            pltpu.SemaphoreType.DMA((N_DEV,)),
            pltpu.SemaphoreType.DMA((N_DEV,)),
            pltpu.SemaphoreType.DMA((N_DEV,)),
            pltpu.SemaphoreType.DMA((N_DEV,)),
        ],
    )(partial, resid, gamma)


# device time: 23644 ns/iter; 1.3136x vs baseline; 1.1669x over previous
import jax
import jax.numpy as jnp
from jax import lax
from jax.experimental import pallas as pl
from jax.experimental.pallas import tpu as pltpu

N_DEV = 16
D = 512
ROWS = D // N_DEV


def kernel(partial, resid, gamma):
    def body(x_ref, resid_ref, gamma_ref, out_ref,
             stage_ref, recv_ref, gather_ref,
             send_sems, recv_sems, send_sems2, recv_sems2):
        my = lax.axis_index("i")

        barrier_sem = pltpu.get_barrier_semaphore()
        for j in range(N_DEV):
            @pl.when(my != j)
            def _(j_=j):
                pl.semaphore_signal(
                    barrier_sem, inc=1,
                    device_id=(j_,), device_id_type=pl.DeviceIdType.MESH,
                )
        pl.semaphore_wait(barrier_sem, N_DEV - 1)

        stage_ref[...] = x_ref[0].astype(jnp.bfloat16).reshape(N_DEV, ROWS, D)

        s1 = []
        for j in range(N_DEV):
            rdma = pltpu.make_async_remote_copy(
                src_ref=stage_ref.at[j],
                dst_ref=recv_ref.at[my],
                send_sem=send_sems.at[j],
                recv_sem=recv_sems.at[my],
                device_id=(j,),
                device_id_type=pl.DeviceIdType.MESH,
            )
            s1.append(rdma)

        for j in range(N_DEV):
            @pl.when(my != j)
            def _(rdma=s1[j]):
                rdma.start()

        recv_ref[pl.ds(my, 1)] = stage_ref[pl.ds(my, 1)]

        for s in range(N_DEV):
            @pl.when(my != s)
            def _(rdma=s1[s]):
                w = pltpu.make_async_remote_copy(
                    src_ref=stage_ref.at[s],
                    dst_ref=recv_ref.at[s],
                    send_sem=send_sems.at[s],
                    recv_sem=recv_sems.at[s],
                    device_id=(s,),
                    device_id_type=pl.DeviceIdType.MESH,
                )
                w.wait_recv()

        acc = jnp.sum(recv_ref[...].astype(jnp.float32), axis=0)
        y = acc + resid_ref[pl.ds(my * ROWS, ROWS), :]
        ms = jnp.mean(y * y, axis=-1, keepdims=True)
        out_chunk = y * lax.rsqrt(ms + 1e-6) * gamma_ref[:][None, :]
        gather_ref[pl.ds(my, 1)] = out_chunk.astype(jnp.bfloat16)[None]

        s2 = []
        for j in range(N_DEV):
            rdma = pltpu.make_async_remote_copy(
                src_ref=gather_ref.at[my],
                dst_ref=gather_ref.at[my],
                send_sem=send_sems2.at[j],
                recv_sem=recv_sems2.at[my],
                device_id=(j,),
                device_id_type=pl.DeviceIdType.MESH,
            )
            s2.append(rdma)

        for j in range(N_DEV):
            @pl.when(my != j)
            def _(rdma=s2[j]):
                rdma.start()

        for j in range(N_DEV):
            @pl.when(my != j)
            def _(rdma=s1[j]):
                rdma.wait_send()

        for s in range(N_DEV):
            @pl.when(my != s)
            def _(s_=s):
                w = pltpu.make_async_remote_copy(
                    src_ref=gather_ref.at[s_],
                    dst_ref=gather_ref.at[s_],
                    send_sem=send_sems2.at[s_],
                    recv_sem=recv_sems2.at[s_],
                    device_id=(s_,),
                    device_id_type=pl.DeviceIdType.MESH,
                )
                w.wait_recv()
        for j in range(N_DEV):
            @pl.when(my != j)
            def _(rdma=s2[j]):
                rdma.wait_send()

        out_ref[...] = gather_ref[...].astype(jnp.float32).reshape(D, D)

    return pl.pallas_call(
        body,
        out_shape=jax.ShapeDtypeStruct((D, D), jnp.float32),
        in_specs=[
            pl.BlockSpec(memory_space=pltpu.VMEM),
            pl.BlockSpec(memory_space=pltpu.VMEM),
            pl.BlockSpec(memory_space=pltpu.VMEM),
        ],
        out_specs=pl.BlockSpec(memory_space=pltpu.VMEM),
        scratch_shapes=[
            pltpu.VMEM((N_DEV, ROWS, D), jnp.bfloat16),
            pltpu.VMEM((N_DEV, ROWS, D), jnp.bfloat16),
            pltpu.VMEM((N_DEV, ROWS, D), jnp.bfloat16),
            pltpu.SemaphoreType.DMA((N_DEV,)),
            pltpu.SemaphoreType.DMA((N_DEV,)),
            pltpu.SemaphoreType.DMA((N_DEV,)),
            pltpu.SemaphoreType.DMA((N_DEV,)),
        ],
        compiler_params=pltpu.CompilerParams(collective_id=0),
    )(partial, resid, gamma)
